# baseline (speedup 1.0000x reference)
import functools

import jax
import jax.numpy as jnp
from jax import lax
from jax.experimental import pallas as pl
from jax.experimental.pallas import tpu as pltpu
from jax.experimental.pallas import tpu_sc as plsc

_N = 10000
_NP = 10240
_E = 320000
_D = 128
_W = 64
_HID = 32
_NTILES = 16
_CHUNK = 128
_NCHUNK = 80
_EHALF = _NTILES * _NCHUNK * _CHUNK
_EP = 2 * _EHALF
_RPT = _NP // _NTILES
_B = 1024
_GRID = _NP // _B
_NBUF = 8


@functools.lru_cache(maxsize=None)
def _sc_degree_fn():
    mesh = plsc.VectorSubcoreMesh(core_axis_name="c", subcore_axis_name="s",
                                  num_cores=1, num_subcores=_NTILES)

    @functools.partial(
        pl.kernel,
        out_type=jax.ShapeDtypeStruct((_NP,), jnp.float32),
        mesh=mesh,
        scratch_types=[
            pltpu.VMEM((_NCHUNK, _CHUNK), jnp.int32),
            pltpu.VMEM((_CHUNK,), jnp.float32),
            pltpu.VMEM_SHARED((_NP,), jnp.float32),
            pltpu.SemaphoreType.DMA,
        ],
    )
    def deg_kernel(src_h, ones_h, zeros_h, deg_out, idx_v, ones_v, deg_sh,
                   sem):
        s = lax.axis_index("s")
        pltpu.sync_copy(zeros_h, deg_sh.at[pl.ds(s * _RPT, _RPT)])
        pltpu.sync_copy(ones_h, ones_v)
        pltpu.sync_copy(src_h.at[s], idx_v)
        plsc.subcore_barrier()

        def fire(j, carry):
            pltpu.async_copy(ones_v, deg_sh.at[idx_v.at[j]], sem, add=True)
            return carry

        lax.fori_loop(0, _NCHUNK, fire, 0)

        def drain(j, carry):
            pltpu.make_async_copy(ones_v, deg_sh.at[idx_v.at[j]], sem).wait()
            return carry

        lax.fori_loop(0, _NCHUNK, drain, 0)
        plsc.subcore_barrier()
        pltpu.sync_copy(deg_sh.at[pl.ds(s * _RPT, _RPT)],
                        deg_out.at[pl.ds(s * _RPT, _RPT)])

    return deg_kernel


@functools.lru_cache(maxsize=None)
def _sc_scatter_fn():
    mesh = plsc.VectorSubcoreMesh(core_axis_name="c", subcore_axis_name="s",
                                  num_cores=1, num_subcores=_NTILES)

    @functools.partial(
        pl.kernel,
        out_type=jax.ShapeDtypeStruct((_NP, _W), jnp.float32),
        mesh=mesh,
        scratch_types=[
            pltpu.VMEM((_NCHUNK, _CHUNK), jnp.int32),
            pltpu.VMEM((_NCHUNK, _CHUNK), jnp.int32),
        ] + [pltpu.VMEM((_CHUNK, _W), jnp.float32) for _ in range(_NBUF)]
          + [pltpu.VMEM_SHARED((_NP, _W), jnp.float32)]
          + [pltpu.SemaphoreType.DMA for _ in range(2 * _NBUF)],
        compiler_params=pltpu.CompilerParams(use_tc_tiling_on_sc=False),
    )
    def scat_kernel(table_h, src_h, dst_h, zrows_h, s_out,
                    sidx, didx, *bufs_and_sems):
        rows = bufs_and_sems[:_NBUF]
        acc_sh = bufs_and_sems[_NBUF]
        gsem = bufs_and_sems[_NBUF + 1:2 * _NBUF + 1]
        ssem = bufs_and_sems[2 * _NBUF + 1:]
        s = lax.axis_index("s")
        pltpu.sync_copy(zrows_h, acc_sh.at[pl.ds(s * _RPT, _RPT)])
        pltpu.sync_copy(src_h.at[s], sidx)
        pltpu.sync_copy(dst_h.at[s], didx)
        plsc.subcore_barrier()

        def gather(j, b):
            pltpu.async_copy(table_h.at[sidx.at[j]], rows[b], gsem[b])

        def gather_wait(j, b):
            pltpu.make_async_copy(table_h.at[sidx.at[j]], rows[b],
                                  gsem[b]).wait()

        def scat(j, b):
            pltpu.async_copy(rows[b], acc_sh.at[didx.at[j]], ssem[b],
                             add=True)

        def scat_wait(j, b):
            pltpu.make_async_copy(rows[b], acc_sh.at[didx.at[j]],
                                  ssem[b]).wait()

        for b in range(_NBUF):
            gather(b, b)

        def grp(g, carry):
            j0 = g * _NBUF
            for b in range(_NBUF):
                gather_wait(j0 + b, b)
                scat(j0 + b, b)
            for b in range(_NBUF):
                scat_wait(j0 + b, b)
                gather(j0 + b + _NBUF, b)
            return carry

        lax.fori_loop(0, _NCHUNK // _NBUF - 1, grp, 0)
        j0 = _NCHUNK - _NBUF
        for b in range(_NBUF):
            gather_wait(j0 + b, b)
            scat(j0 + b, b)
        for b in range(_NBUF):
            scat_wait(j0 + b, b)
        plsc.subcore_barrier()
        pltpu.sync_copy(acc_sh.at[pl.ds(s * _RPT, _RPT)],
                        s_out.at[pl.ds(s * _RPT, _RPT)])

    return scat_kernel


def _sc_degree(src3, ones_h, zeros_h):
    return _sc_degree_fn()(src3, ones_h, zeros_h)


def _sc_scatter(table, src3, dst3, zrows_h):
    return _sc_scatter_fn()(table, src3, dst3, zrows_h)


def _tc1_body(x_ref, dega_ref, degb_ref, w_ref, b_ref,
              c1_ref, p_ref, a_ref, dinv_ref):
    deg = dega_ref[...] + degb_ref[...]
    dinv = jnp.where(deg > 0.0, lax.rsqrt(jnp.maximum(deg, 1e-12)), 0.0)
    y = jnp.dot(x_ref[...], w_ref[...], preferred_element_type=jnp.float32)
    y2 = y[:, 0:_W]
    y1 = y[:, _W:2 * _W]
    y0 = y[:, 2 * _W:3 * _W]
    c1_ref[...] = dinv * y2
    p_ref[...] = dinv * y1
    a_ref[...] = y0 + b_ref[...] - y2
    dinv_ref[...] = dinv


def _tc1(x, dega, degb, wp, bsum):
    row = lambda i: (i, 0)
    zero = lambda i: (0, 0)
    return pl.pallas_call(
        _tc1_body,
        grid=(_GRID,),
        in_specs=[
            pl.BlockSpec((_B, _D), row),
            pl.BlockSpec((_B, 1), row),
            pl.BlockSpec((_B, 1), row),
            pl.BlockSpec((_D, 3 * _W), zero),
            pl.BlockSpec((1, _W), zero),
        ],
        out_specs=[
            pl.BlockSpec((_B, _W), row),
            pl.BlockSpec((_B, _W), row),
            pl.BlockSpec((_B, _W), row),
            pl.BlockSpec((_B, 1), row),
        ],
        out_shape=[
            jax.ShapeDtypeStruct((_NP, _W), jnp.float32),
            jax.ShapeDtypeStruct((_NP, _W), jnp.float32),
            jax.ShapeDtypeStruct((_NP, _W), jnp.float32),
            jax.ShapeDtypeStruct((_NP, 1), jnp.float32),
        ],
    )(x, dega, degb, wp, bsum)


def _tcq_body(s1a_ref, s1b_ref, p_ref, dinv_ref, q_ref):
    dinv = dinv_ref[...]
    q_ref[...] = p_ref[...] - 2.0 * dinv * dinv * (s1a_ref[...] + s1b_ref[...])


def _tc_q(s1a, s1b, p, dinv):
    row = lambda i: (i, 0)
    return pl.pallas_call(
        _tcq_body,
        grid=(_GRID,),
        in_specs=[
            pl.BlockSpec((_B, _W), row),
            pl.BlockSpec((_B, _W), row),
            pl.BlockSpec((_B, _W), row),
            pl.BlockSpec((_B, 1), row),
        ],
        out_specs=pl.BlockSpec((_B, _W), row),
        out_shape=jax.ShapeDtypeStruct((_NP, _W), jnp.float32),
    )(s1a, s1b, p, dinv)


def _gate_h(a, s2a, s2b, dinv):
    cheb = a - dinv * (s2a + s2b)
    z = jax.nn.sigmoid(cheb[:, 0:_HID])
    ht = jnp.tanh(cheb[:, _HID:2 * _HID])
    return jax.nn.relu((1.0 - z) * ht)


def _tc3_body(a_ref, s2a_ref, s2b_ref, dinv_ref, w_ref, b_ref,
              c1_ref, p_ref, a2_ref):
    h = _gate_h(a_ref[...], s2a_ref[...], s2b_ref[...], dinv_ref[...])
    dinv = dinv_ref[...]
    y = jnp.dot(h, w_ref[...], preferred_element_type=jnp.float32)
    y2 = y[:, 0:_W]
    y1 = y[:, _W:2 * _W]
    y0 = y[:, 2 * _W:3 * _W]
    c1_ref[...] = dinv * y2
    p_ref[...] = dinv * y1
    a2_ref[...] = y0 + b_ref[...] - y2


def _tc3(a, s2a, s2b, dinv, wp, bsum):
    row = lambda i: (i, 0)
    zero = lambda i: (0, 0)
    return pl.pallas_call(
        _tc3_body,
        grid=(_GRID,),
        in_specs=[
            pl.BlockSpec((_B, _W), row),
            pl.BlockSpec((_B, _W), row),
            pl.BlockSpec((_B, _W), row),
            pl.BlockSpec((_B, 1), row),
            pl.BlockSpec((_HID, 3 * _W), zero),
            pl.BlockSpec((1, _W), zero),
        ],
        out_specs=[
            pl.BlockSpec((_B, _W), row),
            pl.BlockSpec((_B, _W), row),
            pl.BlockSpec((_B, _W), row),
        ],
        out_shape=[
            jax.ShapeDtypeStruct((_NP, _W), jnp.float32),
            jax.ShapeDtypeStruct((_NP, _W), jnp.float32),
            jax.ShapeDtypeStruct((_NP, _W), jnp.float32),
        ],
    )(a, s2a, s2b, dinv, wp, bsum)


def _tc5_body(a_ref, s2a_ref, s2b_ref, dinv_ref, lw_ref, lb_ref,
              hsum_ref, out_ref):
    i = pl.program_id(0)
    h = _gate_h(a_ref[...], s2a_ref[...], s2b_ref[...], dinv_ref[...])
    rows = i * _B + lax.broadcasted_iota(jnp.int32, (_B, 1), 0)
    h = jnp.where(rows < _N, h, 0.0)
    part = jnp.sum(h, axis=0, keepdims=True)

    @pl.when(i == 0)
    def _():
        hsum_ref[...] = jnp.zeros_like(hsum_ref)
        out_ref[...] = jnp.zeros_like(out_ref)

    hsum_ref[...] += part

    @pl.when(i == pl.num_programs(0) - 1)
    def _():
        pooled = hsum_ref[...] * (1.0 / _N)
        out_ref[...] = (jnp.dot(pooled, lw_ref[...],
                                preferred_element_type=jnp.float32)
                        + lb_ref[...])


def _tc5(a, s2a, s2b, dinv, lw, lb):
    row = lambda i: (i, 0)
    zero = lambda i: (0, 0)
    return pl.pallas_call(
        _tc5_body,
        grid=(_GRID,),
        in_specs=[
            pl.BlockSpec((_B, _W), row),
            pl.BlockSpec((_B, _W), row),
            pl.BlockSpec((_B, _W), row),
            pl.BlockSpec((_B, 1), row),
            pl.BlockSpec((_HID, 11), zero),
            pl.BlockSpec((1, 11), zero),
        ],
        out_specs=[
            pl.BlockSpec((1, _HID), zero),
            pl.BlockSpec((1, 11), zero),
        ],
        out_shape=[
            jax.ShapeDtypeStruct((1, _HID), jnp.float32),
            jax.ShapeDtypeStruct((1, 11), jnp.float32),
        ],
    )(a, s2a, s2b, dinv, lw, lb)


def _pack(wx):
    return jnp.concatenate(
        [wx[0, 2], wx[2, 2], wx[0, 1], wx[2, 1], wx[0, 0], wx[2, 0]], axis=1)


def kernel(obs, edge_index, gru1_Wx, gru1_bx, gru1_Wh, gru1_bh,
           gru2_Wx, gru2_bx, gru2_Wh, gru2_bh, lin_W, lin_b):
    f32 = jnp.float32
    src = edge_index[0]
    dst = edge_index[1]
    padv = _N + (jnp.arange(_EP - _E, dtype=jnp.int32) % (_NP - _N))
    srcp = jnp.concatenate([src, padv]).reshape(2, _NTILES, _NCHUNK, _CHUNK)
    dstp = jnp.concatenate([dst, padv]).reshape(2, _NTILES, _NCHUNK, _CHUNK)
    src_a, src_b = srcp[0], srcp[1]
    dst_a, dst_b = dstp[0], dstp[1]
    obs_p = jnp.zeros((_NP, _D), f32).at[:_N].set(obs)

    wp1 = _pack(gru1_Wx)
    wp2 = _pack(gru2_Wx)
    bsum1 = jnp.concatenate(
        [gru1_bx[0] + gru1_bh[0], gru1_bx[2] + gru1_bh[2]]).reshape(1, _W)
    bsum2 = jnp.concatenate(
        [gru2_bx[0] + gru2_bh[0], gru2_bx[2] + gru2_bh[2]]).reshape(1, _W)

    ones_h = jnp.ones((_CHUNK,), f32)
    zeros_deg = jnp.zeros((_RPT,), f32)
    zeros_rows = jnp.zeros((_RPT, _W), f32)

    dega = _sc_degree(src_a, ones_h, zeros_deg).reshape(_NP, 1)
    degb = _sc_degree(src_b, ones_h, zeros_deg).reshape(_NP, 1)

    def spass(table):
        sa = _sc_scatter(table, src_a, dst_a, zeros_rows)
        sb = _sc_scatter(table, src_b, dst_b, zeros_rows)
        return sa, sb

    c1, p, a, dinv = _tc1(obs_p, dega, degb, wp1, bsum1)
    s1a, s1b = spass(c1)
    q = _tc_q(s1a, s1b, p, dinv)
    s2a, s2b = spass(q)

    c1b, pb, ab = _tc3(a, s2a, s2b, dinv, wp2, bsum2)
    s1c, s1d = spass(c1b)
    qb = _tc_q(s1c, s1d, pb, dinv)
    s2c, s2d = spass(qb)

    _, out11 = _tc5(ab, s2c, s2d, dinv, lin_W, lin_b.reshape(1, 11))
    return out11[0, 1:]


# speedup vs baseline: 1.3862x; 1.3862x over previous
import functools

import jax
import jax.numpy as jnp
from jax import lax
from jax.experimental import pallas as pl
from jax.experimental.pallas import tpu as pltpu
from jax.experimental.pallas import tpu_sc as plsc

_N = 10000
_NP = 10240
_E = 320000
_D = 128
_W = 64
_HID = 32
_NTILES = 32
_CHUNK = 128
_NCHUNK = 80
_NBUF = 8
_EP = _NTILES * _NCHUNK * _CHUNK
_RPT = _NP // 16
_B = 1024
_GRID = _NP // _B


@functools.lru_cache(maxsize=None)
def _sc_degree_fn():
    mesh = plsc.VectorSubcoreMesh(core_axis_name="c", subcore_axis_name="s",
                                  num_cores=2, num_subcores=16)

    @functools.partial(
        pl.kernel,
        out_type=jax.ShapeDtypeStruct((2, _NP), jnp.float32),
        mesh=mesh,
        scratch_types=[
            pltpu.VMEM((_NCHUNK, _CHUNK), jnp.int32),
            pltpu.VMEM((_CHUNK,), jnp.float32),
            pltpu.VMEM_SHARED((_NP,), jnp.float32),
            pltpu.SemaphoreType.DMA,
        ],
    )
    def deg_kernel(src_h, ones_h, zeros_h, deg_out, idx_v, ones_v, deg_sh,
                   sem):
        c = lax.axis_index("c")
        s = lax.axis_index("s")
        wid = c * 16 + s
        pltpu.sync_copy(zeros_h, deg_sh.at[pl.ds(s * _RPT, _RPT)])
        pltpu.sync_copy(ones_h, ones_v)
        pltpu.sync_copy(src_h.at[wid], idx_v)
        plsc.subcore_barrier()

        def fire(j, carry):
            pltpu.async_copy(ones_v, deg_sh.at[idx_v.at[j]], sem, add=True)
            return carry

        lax.fori_loop(0, _NCHUNK, fire, 0)

        def drain(j, carry):
            pltpu.make_async_copy(ones_v, deg_sh.at[idx_v.at[j]], sem).wait()
            return carry

        lax.fori_loop(0, _NCHUNK, drain, 0)
        plsc.subcore_barrier()
        pltpu.sync_copy(deg_sh.at[pl.ds(s * _RPT, _RPT)],
                        deg_out.at[c, pl.ds(s * _RPT, _RPT)])

    return deg_kernel


@functools.lru_cache(maxsize=None)
def _sc_scatter_fn():
    mesh = plsc.VectorSubcoreMesh(core_axis_name="c", subcore_axis_name="s",
                                  num_cores=2, num_subcores=16)

    @functools.partial(
        pl.kernel,
        out_type=jax.ShapeDtypeStruct((2, _NP, _W), jnp.float32),
        mesh=mesh,
        scratch_types=[
            pltpu.VMEM((_NCHUNK, _CHUNK), jnp.int32),
            pltpu.VMEM((_NCHUNK, _CHUNK), jnp.int32),
        ] + [pltpu.VMEM((_CHUNK, _W), jnp.float32) for _ in range(_NBUF)]
          + [pltpu.VMEM_SHARED((_NP, _W), jnp.float32)]
          + [pltpu.SemaphoreType.DMA for _ in range(2 * _NBUF)],
        compiler_params=pltpu.CompilerParams(use_tc_tiling_on_sc=False),
    )
    def scat_kernel(table_h, src_h, dst_h, zrows_h, s_out,
                    sidx, didx, *bufs_and_sems):
        rows = bufs_and_sems[:_NBUF]
        acc_sh = bufs_and_sems[_NBUF]
        gsem = bufs_and_sems[_NBUF + 1:2 * _NBUF + 1]
        ssem = bufs_and_sems[2 * _NBUF + 1:]
        c = lax.axis_index("c")
        s = lax.axis_index("s")
        wid = c * 16 + s
        pltpu.sync_copy(zrows_h, acc_sh.at[pl.ds(s * _RPT, _RPT)])
        pltpu.sync_copy(src_h.at[wid], sidx)
        pltpu.sync_copy(dst_h.at[wid], didx)
        plsc.subcore_barrier()

        def gather(j, b):
            pltpu.async_copy(table_h.at[sidx.at[j]], rows[b], gsem[b])

        def gather_wait(j, b):
            pltpu.make_async_copy(table_h.at[sidx.at[j]], rows[b],
                                  gsem[b]).wait()

        def scat(j, b):
            pltpu.async_copy(rows[b], acc_sh.at[didx.at[j]], ssem[b],
                             add=True)

        def scat_wait(j, b):
            pltpu.make_async_copy(rows[b], acc_sh.at[didx.at[j]],
                                  ssem[b]).wait()

        for b in range(_NBUF):
            gather(b, b)

        def grp(g, carry):
            j0 = g * _NBUF
            for b in range(_NBUF):
                gather_wait(j0 + b, b)
                scat(j0 + b, b)
            for b in range(_NBUF):
                scat_wait(j0 + b, b)
                gather(j0 + b + _NBUF, b)
            return carry

        lax.fori_loop(0, _NCHUNK // _NBUF - 1, grp, 0)
        j0 = _NCHUNK - _NBUF
        for b in range(_NBUF):
            gather_wait(j0 + b, b)
            scat(j0 + b, b)
        for b in range(_NBUF):
            scat_wait(j0 + b, b)
        plsc.subcore_barrier()
        pltpu.sync_copy(acc_sh.at[pl.ds(s * _RPT, _RPT)],
                        s_out.at[c, pl.ds(s * _RPT, _RPT)])

    return scat_kernel


def _sc_degree(src3, ones_h, zeros_h):
    return _sc_degree_fn()(src3, ones_h, zeros_h)


def _sc_scatter(table, src3, dst3, zrows_h):
    return _sc_scatter_fn()(table, src3, dst3, zrows_h)


def _tc1a_body(x_ref, w_ref, b_ref, y2_ref, y1_ref, a_ref):
    y = jnp.dot(x_ref[...], w_ref[...], preferred_element_type=jnp.float32)
    y2 = y[:, 0:_W]
    y2_ref[...] = y2
    y1_ref[...] = y[:, _W:2 * _W]
    a_ref[...] = y[:, 2 * _W:3 * _W] + b_ref[...] - y2


def _tc1a(x, wp, bsum):
    row = lambda i: (i, 0)
    zero = lambda i: (0, 0)
    return pl.pallas_call(
        _tc1a_body,
        grid=(_GRID,),
        in_specs=[
            pl.BlockSpec((_B, _D), row),
            pl.BlockSpec((_D, 3 * _W), zero),
            pl.BlockSpec((1, _W), zero),
        ],
        out_specs=[
            pl.BlockSpec((_B, _W), row),
            pl.BlockSpec((_B, _W), row),
            pl.BlockSpec((_B, _W), row),
        ],
        out_shape=[
            jax.ShapeDtypeStruct((_NP, _W), jnp.float32),
            jax.ShapeDtypeStruct((_NP, _W), jnp.float32),
            jax.ShapeDtypeStruct((_NP, _W), jnp.float32),
        ],
    )(x, wp, bsum)


def _tc1b_body(y2_ref, y1_ref, dega_ref, degb_ref,
               c1_ref, p_ref, dinv_ref):
    deg = dega_ref[...] + degb_ref[...]
    dinv = jnp.where(deg > 0.0, lax.rsqrt(jnp.maximum(deg, 1e-12)), 0.0)
    c1_ref[...] = dinv * y2_ref[...]
    p_ref[...] = dinv * y1_ref[...]
    dinv_ref[...] = dinv


def _tc1b(y2, y1, dega, degb):
    row = lambda i: (i, 0)
    return pl.pallas_call(
        _tc1b_body,
        grid=(_GRID,),
        in_specs=[
            pl.BlockSpec((_B, _W), row),
            pl.BlockSpec((_B, _W), row),
            pl.BlockSpec((_B, 1), row),
            pl.BlockSpec((_B, 1), row),
        ],
        out_specs=[
            pl.BlockSpec((_B, _W), row),
            pl.BlockSpec((_B, _W), row),
            pl.BlockSpec((_B, 1), row),
        ],
        out_shape=[
            jax.ShapeDtypeStruct((_NP, _W), jnp.float32),
            jax.ShapeDtypeStruct((_NP, _W), jnp.float32),
            jax.ShapeDtypeStruct((_NP, 1), jnp.float32),
        ],
    )(y2, y1, dega, degb)


def _tcq_body(s1a_ref, s1b_ref, p_ref, dinv_ref, q_ref):
    dinv = dinv_ref[...]
    q_ref[...] = p_ref[...] - 2.0 * dinv * dinv * (s1a_ref[...] + s1b_ref[...])


def _tc_q(s1a, s1b, p, dinv):
    row = lambda i: (i, 0)
    return pl.pallas_call(
        _tcq_body,
        grid=(_GRID,),
        in_specs=[
            pl.BlockSpec((_B, _W), row),
            pl.BlockSpec((_B, _W), row),
            pl.BlockSpec((_B, _W), row),
            pl.BlockSpec((_B, 1), row),
        ],
        out_specs=pl.BlockSpec((_B, _W), row),
        out_shape=jax.ShapeDtypeStruct((_NP, _W), jnp.float32),
    )(s1a, s1b, p, dinv)


def _gate_h(a, s2a, s2b, dinv):
    cheb = a - dinv * (s2a + s2b)
    z = jax.nn.sigmoid(cheb[:, 0:_HID])
    ht = jnp.tanh(cheb[:, _HID:2 * _HID])
    return jax.nn.relu((1.0 - z) * ht)


def _tc3_body(a_ref, s2a_ref, s2b_ref, dinv_ref, w_ref, b_ref,
              c1_ref, p_ref, a2_ref):
    h = _gate_h(a_ref[...], s2a_ref[...], s2b_ref[...], dinv_ref[...])
    dinv = dinv_ref[...]
    y = jnp.dot(h, w_ref[...], preferred_element_type=jnp.float32)
    y2 = y[:, 0:_W]
    y1 = y[:, _W:2 * _W]
    y0 = y[:, 2 * _W:3 * _W]
    c1_ref[...] = dinv * y2
    p_ref[...] = dinv * y1
    a2_ref[...] = y0 + b_ref[...] - y2


def _tc3(a, s2a, s2b, dinv, wp, bsum):
    row = lambda i: (i, 0)
    zero = lambda i: (0, 0)
    return pl.pallas_call(
        _tc3_body,
        grid=(_GRID,),
        in_specs=[
            pl.BlockSpec((_B, _W), row),
            pl.BlockSpec((_B, _W), row),
            pl.BlockSpec((_B, _W), row),
            pl.BlockSpec((_B, 1), row),
            pl.BlockSpec((_HID, 3 * _W), zero),
            pl.BlockSpec((1, _W), zero),
        ],
        out_specs=[
            pl.BlockSpec((_B, _W), row),
            pl.BlockSpec((_B, _W), row),
            pl.BlockSpec((_B, _W), row),
        ],
        out_shape=[
            jax.ShapeDtypeStruct((_NP, _W), jnp.float32),
            jax.ShapeDtypeStruct((_NP, _W), jnp.float32),
            jax.ShapeDtypeStruct((_NP, _W), jnp.float32),
        ],
    )(a, s2a, s2b, dinv, wp, bsum)


def _tc5_body(a_ref, s2a_ref, s2b_ref, dinv_ref, lw_ref, lb_ref,
              hsum_ref, out_ref):
    i = pl.program_id(0)
    h = _gate_h(a_ref[...], s2a_ref[...], s2b_ref[...], dinv_ref[...])
    rows = i * _B + lax.broadcasted_iota(jnp.int32, (_B, 1), 0)
    h = jnp.where(rows < _N, h, 0.0)
    part = jnp.sum(h, axis=0, keepdims=True)

    @pl.when(i == 0)
    def _():
        hsum_ref[...] = jnp.zeros_like(hsum_ref)
        out_ref[...] = jnp.zeros_like(out_ref)

    hsum_ref[...] += part

    @pl.when(i == pl.num_programs(0) - 1)
    def _():
        pooled = hsum_ref[...] * (1.0 / _N)
        out_ref[...] = (jnp.dot(pooled, lw_ref[...],
                                preferred_element_type=jnp.float32)
                        + lb_ref[...])


def _tc5(a, s2a, s2b, dinv, lw, lb):
    row = lambda i: (i, 0)
    zero = lambda i: (0, 0)
    return pl.pallas_call(
        _tc5_body,
        grid=(_GRID,),
        in_specs=[
            pl.BlockSpec((_B, _W), row),
            pl.BlockSpec((_B, _W), row),
            pl.BlockSpec((_B, _W), row),
            pl.BlockSpec((_B, 1), row),
            pl.BlockSpec((_HID, 11), zero),
            pl.BlockSpec((1, 11), zero),
        ],
        out_specs=[
            pl.BlockSpec((1, _HID), zero),
            pl.BlockSpec((1, 11), zero),
        ],
        out_shape=[
            jax.ShapeDtypeStruct((1, _HID), jnp.float32),
            jax.ShapeDtypeStruct((1, 11), jnp.float32),
        ],
    )(a, s2a, s2b, dinv, lw, lb)


def _pack(wx):
    return jnp.concatenate(
        [wx[0, 2], wx[2, 2], wx[0, 1], wx[2, 1], wx[0, 0], wx[2, 0]], axis=1)


def kernel(obs, edge_index, gru1_Wx, gru1_bx, gru1_Wh, gru1_bh,
           gru2_Wx, gru2_bx, gru2_Wh, gru2_bh, lin_W, lin_b):
    f32 = jnp.float32
    src = edge_index[0]
    dst = edge_index[1]
    padv = _N + (jnp.arange(_EP - _E, dtype=jnp.int32) % (_NP - _N))
    src3 = jnp.concatenate([src, padv]).reshape(_NTILES, _NCHUNK, _CHUNK)
    dst3 = jnp.concatenate([dst, padv]).reshape(_NTILES, _NCHUNK, _CHUNK)
    obs_p = jnp.zeros((_NP, _D), f32).at[:_N].set(obs)

    wp1 = _pack(gru1_Wx)
    wp2 = _pack(gru2_Wx)
    bsum1 = jnp.concatenate(
        [gru1_bx[0] + gru1_bh[0], gru1_bx[2] + gru1_bh[2]]).reshape(1, _W)
    bsum2 = jnp.concatenate(
        [gru2_bx[0] + gru2_bh[0], gru2_bx[2] + gru2_bh[2]]).reshape(1, _W)

    ones_h = jnp.ones((_CHUNK,), f32)
    zeros_deg = jnp.zeros((_RPT,), f32)
    zeros_rows = jnp.zeros((_RPT, _W), f32)

    deg2 = _sc_degree(src3, ones_h, zeros_deg)
    ya2, ya1, a = _tc1a(obs_p, wp1, bsum1)
    dega = deg2[0].reshape(_NP, 1)
    degb = deg2[1].reshape(_NP, 1)

    c1, p, dinv = _tc1b(ya2, ya1, dega, degb)
    s1 = _sc_scatter(c1, src3, dst3, zeros_rows)
    q = _tc_q(s1[0], s1[1], p, dinv)
    s2 = _sc_scatter(q, src3, dst3, zeros_rows)

    c1b, pb, ab = _tc3(a, s2[0], s2[1], dinv, wp2, bsum2)
    s1b = _sc_scatter(c1b, src3, dst3, zeros_rows)
    qb = _tc_q(s1b[0], s1b[1], pb, dinv)
    s2b = _sc_scatter(qb, src3, dst3, zeros_rows)

    _, out11 = _tc5(ab, s2b[0], s2b[1], dinv, lin_W, lin_b.reshape(1, 11))
    return out11[0, 1:]


# speedup vs baseline: 1.4233x; 1.0268x over previous
import functools

import jax
import jax.numpy as jnp
from jax import lax
from jax.experimental import pallas as pl
from jax.experimental.pallas import tpu as pltpu
from jax.experimental.pallas import tpu_sc as plsc

_N = 10000
_NP = 10240
_E = 320000
_D = 128
_W = 64
_HID = 32
_NTILES = 32
_CHUNK = 128
_NCHUNK = 80
_NBUF = 8
_EP = _NTILES * _NCHUNK * _CHUNK
_RPT = _NP // 16
_B = 1024
_GRID = _NP // _B


@functools.lru_cache(maxsize=None)
def _sc_degree_fn():
    mesh = plsc.VectorSubcoreMesh(core_axis_name="c", subcore_axis_name="s",
                                  num_cores=2, num_subcores=16)

    @functools.partial(
        pl.kernel,
        out_type=jax.ShapeDtypeStruct((2, _NP), jnp.float32),
        mesh=mesh,
        scratch_types=[
            pltpu.VMEM((_NCHUNK, _CHUNK), jnp.int32),
            pltpu.VMEM((_CHUNK,), jnp.float32),
            pltpu.VMEM_SHARED((_NP,), jnp.float32),
            pltpu.SemaphoreType.DMA,
        ],
    )
    def deg_kernel(src_h, ones_h, zeros_h, deg_out, idx_v, ones_v, deg_sh,
                   sem):
        c = lax.axis_index("c")
        s = lax.axis_index("s")
        wid = c * 16 + s
        pltpu.sync_copy(zeros_h, deg_sh.at[pl.ds(s * _RPT, _RPT)])
        pltpu.sync_copy(ones_h, ones_v)
        pltpu.sync_copy(src_h.at[wid], idx_v)
        plsc.subcore_barrier()

        def fire(j, carry):
            pltpu.async_copy(ones_v, deg_sh.at[idx_v.at[j]], sem, add=True)
            return carry

        lax.fori_loop(0, _NCHUNK, fire, 0)

        def drain(j, carry):
            pltpu.make_async_copy(ones_v, deg_sh.at[idx_v.at[j]], sem).wait()
            return carry

        lax.fori_loop(0, _NCHUNK, drain, 0)
        plsc.subcore_barrier()
        pltpu.sync_copy(deg_sh.at[pl.ds(s * _RPT, _RPT)],
                        deg_out.at[c, pl.ds(s * _RPT, _RPT)])

    return deg_kernel


@functools.lru_cache(maxsize=None)
def _sc_scatter_fn():
    mesh = plsc.VectorSubcoreMesh(core_axis_name="c", subcore_axis_name="s",
                                  num_cores=2, num_subcores=16)

    @functools.partial(
        pl.kernel,
        out_type=jax.ShapeDtypeStruct((2, _NP, _W), jnp.float32),
        mesh=mesh,
        scratch_types=[
            pltpu.VMEM((_NCHUNK, _CHUNK), jnp.int32),
            pltpu.VMEM((_NCHUNK, _CHUNK), jnp.int32),
        ] + [pltpu.VMEM((_CHUNK, _W), jnp.float32) for _ in range(_NBUF)]
          + [pltpu.VMEM_SHARED((_NP, _W), jnp.float32)]
          + [pltpu.SemaphoreType.DMA for _ in range(2 * _NBUF)],
        compiler_params=pltpu.CompilerParams(use_tc_tiling_on_sc=False),
    )
    def scat_kernel(table_h, src_h, dst_h, zrows_h, s_out,
                    sidx, didx, *bufs_and_sems):
        rows = bufs_and_sems[:_NBUF]
        acc_sh = bufs_and_sems[_NBUF]
        gsem = bufs_and_sems[_NBUF + 1:2 * _NBUF + 1]
        ssem = bufs_and_sems[2 * _NBUF + 1:]
        c = lax.axis_index("c")
        s = lax.axis_index("s")
        wid = c * 16 + s
        cz = pltpu.async_copy(zrows_h, acc_sh.at[pl.ds(s * _RPT, _RPT)],
                              gsem[0])
        cs = pltpu.async_copy(src_h.at[wid], sidx, gsem[1])
        cd = pltpu.async_copy(dst_h.at[wid], didx, gsem[2])
        cz.wait()
        cs.wait()
        cd.wait()
        plsc.subcore_barrier()

        def gather(j, b):
            pltpu.async_copy(table_h.at[sidx.at[j]], rows[b], gsem[b])

        def gather_wait(j, b):
            pltpu.make_async_copy(table_h.at[sidx.at[j]], rows[b],
                                  gsem[b]).wait()

        def scat(j, b):
            pltpu.async_copy(rows[b], acc_sh.at[didx.at[j]], ssem[b],
                             add=True)

        def scat_wait(j, b):
            pltpu.make_async_copy(rows[b], acc_sh.at[didx.at[j]],
                                  ssem[b]).wait()

        for b in range(_NBUF):
            gather(b, b)

        def grp(g, carry):
            j0 = g * _NBUF
            for b in range(_NBUF):
                gather_wait(j0 + b, b)
                scat(j0 + b, b)
            for b in range(_NBUF):
                scat_wait(j0 + b, b)
                gather(j0 + b + _NBUF, b)
            return carry

        lax.fori_loop(0, _NCHUNK // _NBUF - 1, grp, 0)
        j0 = _NCHUNK - _NBUF
        for b in range(_NBUF):
            gather_wait(j0 + b, b)
            scat(j0 + b, b)
        for b in range(_NBUF):
            scat_wait(j0 + b, b)
        plsc.subcore_barrier()
        pltpu.sync_copy(acc_sh.at[pl.ds(s * _RPT, _RPT)],
                        s_out.at[c, pl.ds(s * _RPT, _RPT)])

    return scat_kernel


def _sc_degree(src3, ones_h, zeros_h):
    return _sc_degree_fn()(src3, ones_h, zeros_h)


def _sc_scatter(table, src3, dst3, zrows_h):
    return _sc_scatter_fn()(table, src3, dst3, zrows_h)


def _tc1_body(x_ref, dega_ref, degb_ref, w_ref, b_ref,
              c1_ref, p_ref, a_ref, dinv_ref):
    deg = dega_ref[...] + degb_ref[...]
    dinv = jnp.where(deg > 0.0, lax.rsqrt(jnp.maximum(deg, 1e-12)), 0.0)
    y = jnp.dot(x_ref[...], w_ref[...], preferred_element_type=jnp.float32)
    y2 = y[:, 0:_W]
    y1 = y[:, _W:2 * _W]
    y0 = y[:, 2 * _W:3 * _W]
    c1_ref[...] = dinv * y2
    p_ref[...] = dinv * y1
    a_ref[...] = y0 + b_ref[...] - y2
    dinv_ref[...] = dinv


def _tc1(x, dega, degb, wp, bsum):
    row = lambda i: (i, 0)
    zero = lambda i: (0, 0)
    return pl.pallas_call(
        _tc1_body,
        grid=(_GRID,),
        in_specs=[
            pl.BlockSpec((_B, _D), row),
            pl.BlockSpec((_B, 1), row),
            pl.BlockSpec((_B, 1), row),
            pl.BlockSpec((_D, 3 * _W), zero),
            pl.BlockSpec((1, _W), zero),
        ],
        out_specs=[
            pl.BlockSpec((_B, _W), row),
            pl.BlockSpec((_B, _W), row),
            pl.BlockSpec((_B, _W), row),
            pl.BlockSpec((_B, 1), row),
        ],
        out_shape=[
            jax.ShapeDtypeStruct((_NP, _W), jnp.float32),
            jax.ShapeDtypeStruct((_NP, _W), jnp.float32),
            jax.ShapeDtypeStruct((_NP, _W), jnp.float32),
            jax.ShapeDtypeStruct((_NP, 1), jnp.float32),
        ],
    )(x, dega, degb, wp, bsum)


def _tcq_body(s1a_ref, s1b_ref, p_ref, dinv_ref, q_ref):
    dinv = dinv_ref[...]
    q_ref[...] = p_ref[...] - 2.0 * dinv * dinv * (s1a_ref[...] + s1b_ref[...])


def _tc_q(s1a, s1b, p, dinv):
    row = lambda i: (i, 0)
    return pl.pallas_call(
        _tcq_body,
        grid=(_GRID,),
        in_specs=[
            pl.BlockSpec((_B, _W), row),
            pl.BlockSpec((_B, _W), row),
            pl.BlockSpec((_B, _W), row),
            pl.BlockSpec((_B, 1), row),
        ],
        out_specs=pl.BlockSpec((_B, _W), row),
        out_shape=jax.ShapeDtypeStruct((_NP, _W), jnp.float32),
    )(s1a, s1b, p, dinv)


def _gate_h(a, s2a, s2b, dinv):
    cheb = a - dinv * (s2a + s2b)
    z = jax.nn.sigmoid(cheb[:, 0:_HID])
    ht = jnp.tanh(cheb[:, _HID:2 * _HID])
    return jax.nn.relu((1.0 - z) * ht)


def _tc3_body(a_ref, s2a_ref, s2b_ref, dinv_ref, w_ref, b_ref,
              c1_ref, p_ref, a2_ref):
    h = _gate_h(a_ref[...], s2a_ref[...], s2b_ref[...], dinv_ref[...])
    dinv = dinv_ref[...]
    y = jnp.dot(h, w_ref[...], preferred_element_type=jnp.float32)
    y2 = y[:, 0:_W]
    y1 = y[:, _W:2 * _W]
    y0 = y[:, 2 * _W:3 * _W]
    c1_ref[...] = dinv * y2
    p_ref[...] = dinv * y1
    a2_ref[...] = y0 + b_ref[...] - y2


def _tc3(a, s2a, s2b, dinv, wp, bsum):
    row = lambda i: (i, 0)
    zero = lambda i: (0, 0)
    return pl.pallas_call(
        _tc3_body,
        grid=(_GRID,),
        in_specs=[
            pl.BlockSpec((_B, _W), row),
            pl.BlockSpec((_B, _W), row),
            pl.BlockSpec((_B, _W), row),
            pl.BlockSpec((_B, 1), row),
            pl.BlockSpec((_HID, 3 * _W), zero),
            pl.BlockSpec((1, _W), zero),
        ],
        out_specs=[
            pl.BlockSpec((_B, _W), row),
            pl.BlockSpec((_B, _W), row),
            pl.BlockSpec((_B, _W), row),
        ],
        out_shape=[
            jax.ShapeDtypeStruct((_NP, _W), jnp.float32),
            jax.ShapeDtypeStruct((_NP, _W), jnp.float32),
            jax.ShapeDtypeStruct((_NP, _W), jnp.float32),
        ],
    )(a, s2a, s2b, dinv, wp, bsum)


def _tc5_body(a_ref, s2a_ref, s2b_ref, dinv_ref, lw_ref, lb_ref,
              hsum_ref, out_ref):
    i = pl.program_id(0)
    h = _gate_h(a_ref[...], s2a_ref[...], s2b_ref[...], dinv_ref[...])
    rows = i * _B + lax.broadcasted_iota(jnp.int32, (_B, 1), 0)
    h = jnp.where(rows < _N, h, 0.0)
    part = jnp.sum(h, axis=0, keepdims=True)

    @pl.when(i == 0)
    def _():
        hsum_ref[...] = jnp.zeros_like(hsum_ref)
        out_ref[...] = jnp.zeros_like(out_ref)

    hsum_ref[...] += part

    @pl.when(i == pl.num_programs(0) - 1)
    def _():
        pooled = hsum_ref[...] * (1.0 / _N)
        out_ref[...] = (jnp.dot(pooled, lw_ref[...],
                                preferred_element_type=jnp.float32)
                        + lb_ref[...])


def _tc5(a, s2a, s2b, dinv, lw, lb):
    row = lambda i: (i, 0)
    zero = lambda i: (0, 0)
    return pl.pallas_call(
        _tc5_body,
        grid=(_GRID,),
        in_specs=[
            pl.BlockSpec((_B, _W), row),
            pl.BlockSpec((_B, _W), row),
            pl.BlockSpec((_B, _W), row),
            pl.BlockSpec((_B, 1), row),
            pl.BlockSpec((_HID, 11), zero),
            pl.BlockSpec((1, 11), zero),
        ],
        out_specs=[
            pl.BlockSpec((1, _HID), zero),
            pl.BlockSpec((1, 11), zero),
        ],
        out_shape=[
            jax.ShapeDtypeStruct((1, _HID), jnp.float32),
            jax.ShapeDtypeStruct((1, 11), jnp.float32),
        ],
    )(a, s2a, s2b, dinv, lw, lb)


def _pack(wx):
    return jnp.concatenate(
        [wx[0, 2], wx[2, 2], wx[0, 1], wx[2, 1], wx[0, 0], wx[2, 0]], axis=1)


def kernel(obs, edge_index, gru1_Wx, gru1_bx, gru1_Wh, gru1_bh,
           gru2_Wx, gru2_bx, gru2_Wh, gru2_bh, lin_W, lin_b):
    f32 = jnp.float32
    src = edge_index[0]
    dst = edge_index[1]
    padv = _N + (jnp.arange(_EP - _E, dtype=jnp.int32) % (_NP - _N))
    src3 = jnp.concatenate([src, padv]).reshape(_NTILES, _NCHUNK, _CHUNK)
    dst3 = jnp.concatenate([dst, padv]).reshape(_NTILES, _NCHUNK, _CHUNK)
    obs_p = jnp.zeros((_NP, _D), f32).at[:_N].set(obs)

    wp1 = _pack(gru1_Wx)
    wp2 = _pack(gru2_Wx)
    bsum1 = jnp.concatenate(
        [gru1_bx[0] + gru1_bh[0], gru1_bx[2] + gru1_bh[2]]).reshape(1, _W)
    bsum2 = jnp.concatenate(
        [gru2_bx[0] + gru2_bh[0], gru2_bx[2] + gru2_bh[2]]).reshape(1, _W)

    ones_h = jnp.ones((_CHUNK,), f32)
    zeros_deg = jnp.zeros((_RPT,), f32)
    zeros_rows = jnp.zeros((_RPT, _W), f32)

    deg2 = _sc_degree(src3, ones_h, zeros_deg)
    dega = deg2[0].reshape(_NP, 1)
    degb = deg2[1].reshape(_NP, 1)

    c1, p, a, dinv = _tc1(obs_p, dega, degb, wp1, bsum1)
    s1 = _sc_scatter(c1, src3, dst3, zeros_rows)
    q = _tc_q(s1[0], s1[1], p, dinv)
    s2 = _sc_scatter(q, src3, dst3, zeros_rows)

    c1b, pb, ab = _tc3(a, s2[0], s2[1], dinv, wp2, bsum2)
    s1b = _sc_scatter(c1b, src3, dst3, zeros_rows)
    qb = _tc_q(s1b[0], s1b[1], pb, dinv)
    s2b = _sc_scatter(qb, src3, dst3, zeros_rows)

    _, out11 = _tc5(ab, s2b[0], s2b[1], dinv, lin_W, lin_b.reshape(1, 11))
    return out11[0, 1:]
